# baseline (speedup 1.0000x reference)
import functools

import jax
import jax.numpy as jnp
from jax import lax
from jax.experimental import pallas as pl
from jax.experimental.pallas import tpu as pltpu
from jax.experimental.pallas import tpu_sc as plsc

N = 10000
E = 160000
IN_CH = 256
H1 = 512
H2 = 256
DEPTH = 4

NC, NS, L = 2, 16, 16
NW = NC * NS
Nt = 10240
CHUNK = 128
NCH = 80
EPAD = NS * CHUNK * NCH
EPT = EPAD // NS
EPW = EPAD // NW
ROWS_PT = Nt // NS
BN = 512

_MESH = plsc.VectorSubcoreMesh(core_axis_name="c", subcore_axis_name="s",
                               num_cores=NC, num_subcores=NS)
_SC_PARAMS = pltpu.CompilerParams(needs_layout_passes=False,
                                  use_tc_tiling_on_sc=True)


def _hist_body(dst_hbm, out_hbm, hist_v, didx, sem):
    c = lax.axis_index("c")
    s = lax.axis_index("s")
    wid = s * NC + c
    zeros16 = jnp.zeros((L,), jnp.float32)
    ones16 = jnp.ones((L,), jnp.float32)

    def zero_body(i, _):
        hist_v[pl.ds(i * L, L)] = zeros16
        return 0
    lax.fori_loop(0, Nt // L, zero_body, 0)

    pltpu.sync_copy(dst_hbm.at[wid], didx)

    def body(i, _):
        for j in range(4):
            idx16 = didx[pl.ds(i * 4 * L + j * L, L)]
            plsc.addupdate_scatter(hist_v, [idx16], ones16)
        return 0
    lax.fori_loop(0, EPW // (4 * L), body, 0)

    pltpu.sync_copy(hist_v, out_hbm.at[wid])


def _sc_histogram(dst2):
    k = functools.partial(
        pl.kernel,
        out_type=jax.ShapeDtypeStruct((NW, Nt), jnp.float32),
        mesh=_MESH,
        scratch_types=[
            pltpu.VMEM((Nt,), jnp.float32),
            pltpu.VMEM((EPW,), jnp.int32),
            pltpu.SemaphoreType.DMA,
        ],
        compiler_params=_SC_PARAMS,
    )(_hist_body)
    return k(dst2)


def _spmm_body(cbn, u_hbm, src_hbm, dst_hbm, out_hbm, sidx, didx, rows,
               acc, sem0, sem1):
    c = lax.axis_index("c")
    s = lax.axis_index("s")
    npass = cbn // NC
    zeros16 = jnp.zeros((L,), jnp.float32)
    sems = (sem0, sem1)

    def zero_rows_body(i, _):
        for j in range(128 // L):
            rows[0, i, pl.ds(j * L, L)] = zeros16
        return 0

    def gather(chunk, buf):
        pltpu.async_copy(u_hbm.at[sidx.at[chunk]], rows.at[buf], sems[buf])

    def gather_wait(chunk, buf):
        pltpu.make_async_copy(u_hbm.at[sidx.at[chunk]], rows.at[buf],
                              sems[buf]).wait()

    def scatter(chunk, buf):
        pltpu.sync_copy(rows.at[buf], acc.at[didx.at[chunk]], add=True)

    def stage(q, cb):
        pltpu.sync_copy(src_hbm.at[cb * NS * 2 + s * 2 + q], sidx)
        pltpu.sync_copy(dst_hbm.at[s * 2 + q], didx)

    def pipeline(be, bo):
        def edge_body(i2, _):
            a = 2 * i2
            gather(a + 1, bo)
            gather_wait(a, be)
            scatter(a, be)

            @pl.when(a + 2 < NCH // 2)
            def _():
                gather(a + 2, be)
            gather_wait(a + 1, bo)
            scatter(a + 1, bo)
            return 0
        lax.fori_loop(0, NCH // 4, edge_body, 0)

    def drain(cb):
        r0 = s * ROWS_PT
        pltpu.sync_copy(acc.at[pl.ds(r0, ROWS_PT)],
                        out_hbm.at[pl.ds(cb * Nt + r0, ROWS_PT)])

    for p in range(npass):
        cb = c * npass + p
        stage(0, cb)
        gather(0, 1)
        lax.fori_loop(0, CHUNK, zero_rows_body, 0)
        if p > 0:
            drain(cb - 1)
        for z in range(ROWS_PT // CHUNK):
            pltpu.sync_copy(rows.at[0],
                            acc.at[pl.ds(s * ROWS_PT + z * CHUNK, CHUNK)])
        plsc.subcore_barrier()
        pipeline(1, 0)
        stage(1, cb)
        gather(0, 0)
        pipeline(0, 1)
        plsc.subcore_barrier()

    drain(c * npass + npass - 1)


def _sc_spmm(u2d, src3, dst3, cbn):
    k = functools.partial(
        pl.kernel,
        out_type=jax.ShapeDtypeStruct((cbn * Nt, 128), jnp.float32),
        mesh=_MESH,
        scratch_types=[
            pltpu.VMEM((NCH // 2, CHUNK), jnp.int32),
            pltpu.VMEM((NCH // 2, CHUNK), jnp.int32),
            pltpu.VMEM((2, CHUNK, 128), jnp.float32),
            pltpu.VMEM_SHARED((Nt, 128), jnp.float32),
            pltpu.SemaphoreType.DMA,
            pltpu.SemaphoreType.DMA,
        ],
        compiler_params=_SC_PARAMS,
    )(functools.partial(_spmm_body, cbn))
    return k(u2d, src3, dst3)


def _prep_kernel(hist_ref, x_ref, u_ref, dinv_ref):
    deg = jnp.sum(hist_ref[...], axis=0) + 1.0
    rb = pl.program_id(0)
    row = rb * BN + lax.broadcasted_iota(jnp.int32, (BN, 1), 0)
    dinv = jnp.where(row < N, lax.rsqrt(deg.reshape(BN, 1)), 0.0)
    dinv_ref[...] = dinv
    for cb in range(IN_CH // 128):
        u_ref[cb] = x_ref[:, cb * 128:(cb + 1) * 128] * dinv


def _tc_prep(hist, xp):
    grid = (Nt // BN,)
    return pl.pallas_call(
        _prep_kernel,
        grid=grid,
        in_specs=[
            pl.BlockSpec((NW, BN), lambda r: (0, r)),
            pl.BlockSpec((BN, IN_CH), lambda r: (r, 0)),
        ],
        out_specs=[
            pl.BlockSpec((IN_CH // 128, BN, 128), lambda r: (0, r, 0)),
            pl.BlockSpec((BN, 1), lambda r: (r, 0)),
        ],
        out_shape=[
            jax.ShapeDtypeStruct((IN_CH // 128, Nt, 128), jnp.float32),
            jax.ShapeDtypeStruct((Nt, 1), jnp.float32),
        ],
    )(hist, xp)


def _stage_kernel(kbi, kbo, residual, g_ref, u_ref, w_ref, b_ref, dinv_ref,
                  uo_ref):
    acc = jnp.zeros((BN, kbo * 128), jnp.float32)
    for kb in range(kbi):
        t = (g_ref[kb] + u_ref[kb]).astype(jnp.bfloat16)
        w = w_ref[kb * 128:(kb + 1) * 128, :].astype(jnp.bfloat16)
        acc = acc + jnp.dot(t, w, preferred_element_type=jnp.float32)
    dinv = dinv_ref[...]
    z = jnp.maximum(acc * dinv + b_ref[...], 0.0)
    for ob in range(kbo):
        un = z[:, ob * 128:(ob + 1) * 128] * dinv
        if residual:
            un = un + u_ref[ob]
        uo_ref[ob] = un


def _tc_stage(g3, u3, W, b, dinv, kbi, kbo):
    residual = kbi == kbo
    grid = (Nt // BN,)
    blk3i = pl.BlockSpec((kbi, BN, 128), lambda r: (0, r, 0))
    blk3o = pl.BlockSpec((kbo, BN, 128), lambda r: (0, r, 0))
    in_specs = [
        blk3i, blk3i,
        pl.BlockSpec((kbi * 128, kbo * 128), lambda r: (0, 0)),
        pl.BlockSpec((1, kbo * 128), lambda r: (0, 0)),
        pl.BlockSpec((BN, 1), lambda r: (r, 0)),
    ]
    return pl.pallas_call(
        functools.partial(_stage_kernel, kbi, kbo, residual),
        grid=grid,
        in_specs=in_specs,
        out_specs=blk3o,
        out_shape=jax.ShapeDtypeStruct((kbo, Nt, 128), jnp.float32),
    )(g3, u3, W, b.reshape(1, -1), dinv)


def _final_kernel(g_ref, u_ref, wmu_ref, bmu_ref, wls_ref, bls_ref, dinv_ref,
                  mu_ref, ls_ref):
    kbi = H1 // 128
    accmu = jnp.zeros((BN, H2), jnp.float32)
    accls = jnp.zeros((BN, H2), jnp.float32)
    for kb in range(kbi):
        t = (g_ref[kb] + u_ref[kb]).astype(jnp.bfloat16)
        accmu = accmu + jnp.dot(
            t, wmu_ref[kb * 128:(kb + 1) * 128, :].astype(jnp.bfloat16),
            preferred_element_type=jnp.float32)
        accls = accls + jnp.dot(
            t, wls_ref[kb * 128:(kb + 1) * 128, :].astype(jnp.bfloat16),
            preferred_element_type=jnp.float32)
    dinv = dinv_ref[...]
    mu_ref[...] = accmu * dinv + bmu_ref[...]
    ls_ref[...] = accls * dinv + bls_ref[...]


def _tc_final(g3, u3, Wmu, bmu, Wls, bls, dinv):
    kbi = H1 // 128
    grid = (Nt // BN,)
    blk3 = pl.BlockSpec((kbi, BN, 128), lambda r: (0, r, 0))
    return pl.pallas_call(
        _final_kernel,
        grid=grid,
        in_specs=[
            blk3, blk3,
            pl.BlockSpec((H1, H2), lambda r: (0, 0)),
            pl.BlockSpec((1, H2), lambda r: (0, 0)),
            pl.BlockSpec((H1, H2), lambda r: (0, 0)),
            pl.BlockSpec((1, H2), lambda r: (0, 0)),
            pl.BlockSpec((BN, 1), lambda r: (r, 0)),
        ],
        out_specs=[
            pl.BlockSpec((BN, H2), lambda r: (r, 0)),
            pl.BlockSpec((BN, H2), lambda r: (r, 0)),
        ],
        out_shape=[
            jax.ShapeDtypeStruct((Nt, H2), jnp.float32),
            jax.ShapeDtypeStruct((Nt, H2), jnp.float32),
        ],
    )(g3, u3, Wmu, bmu.reshape(1, -1), Wls, bls.reshape(1, -1), dinv)


def kernel(x, edge_index, W1, b1, Wres, bres, Wmu, bmu, Wls, bls):
    npad = EPAD - E
    pad = (N + (jnp.arange(npad, dtype=jnp.int32) % 128)).astype(jnp.int32)
    srcp = jnp.concatenate([edge_index[0], pad])
    dstp = jnp.concatenate([edge_index[1], pad])
    cbs4 = (jnp.arange(4, dtype=jnp.int32) * Nt)[:, None]
    src3_4 = (srcp[None, :] + cbs4).reshape(4 * NS * 2, NCH // 2, CHUNK)
    src3_2 = (srcp[None, :] + cbs4[:2]).reshape(2 * NS * 2, NCH // 2, CHUNK)
    dst3 = dstp.reshape(NS * 2, NCH // 2, CHUNK)
    dst2 = dstp.reshape(NW, EPW)
    xp = jnp.pad(x, ((0, Nt - N), (0, 0)))

    hist = _sc_histogram(dst2)
    u, dinv = _tc_prep(hist, xp)

    kb1 = IN_CH // 128
    kbh = H1 // 128
    g = _sc_spmm(u.reshape(kb1 * Nt, 128), src3_2, dst3, kb1).reshape(kb1, Nt, 128)
    u = _tc_stage(g, u, W1, b1, dinv, kb1, kbh)
    for _ in range(DEPTH):
        g = _sc_spmm(u.reshape(kbh * Nt, 128), src3_4, dst3, kbh).reshape(kbh, Nt, 128)
        u = _tc_stage(g, u, Wres, bres, dinv, kbh, kbh)
    g = _sc_spmm(u.reshape(kbh * Nt, 128), src3_4, dst3, kbh).reshape(kbh, Nt, 128)
    mu, ls = _tc_final(g, u, Wmu, bmu, Wls, bls, dinv)
    return mu[:N], ls[:N]


# speedup vs baseline: 1.0037x; 1.0037x over previous
import functools

import jax
import jax.numpy as jnp
from jax import lax
from jax.experimental import pallas as pl
from jax.experimental.pallas import tpu as pltpu
from jax.experimental.pallas import tpu_sc as plsc

N = 10000
E = 160000
IN_CH = 256
H1 = 512
H2 = 256
DEPTH = 4

NC, NS, L = 2, 16, 16
NW = NC * NS
Nt = 10240
CHUNK = 128
NCH = 80
EPAD = NS * CHUNK * NCH
EPT = EPAD // NS
EPW = EPAD // NW
ROWS_PT = Nt // NS
BN = 512

_MESH = plsc.VectorSubcoreMesh(core_axis_name="c", subcore_axis_name="s",
                               num_cores=NC, num_subcores=NS)
_SC_PARAMS = pltpu.CompilerParams(needs_layout_passes=False)


def _hist_body(dst_hbm, out_hbm, hist_v, didx, sem):
    c = lax.axis_index("c")
    s = lax.axis_index("s")
    wid = s * NC + c
    zeros16 = jnp.zeros((L,), jnp.float32)
    ones16 = jnp.ones((L,), jnp.float32)

    def zero_body(i, _):
        hist_v[pl.ds(i * L, L)] = zeros16
        return 0
    lax.fori_loop(0, Nt // L, zero_body, 0)

    pltpu.sync_copy(dst_hbm.at[wid], didx)

    def body(i, _):
        for j in range(4):
            idx16 = didx[pl.ds(i * 4 * L + j * L, L)]
            plsc.addupdate_scatter(hist_v, [idx16], ones16)
        return 0
    lax.fori_loop(0, EPW // (4 * L), body, 0)

    pltpu.sync_copy(hist_v, out_hbm.at[wid])


def _sc_histogram(dst2):
    k = functools.partial(
        pl.kernel,
        out_type=jax.ShapeDtypeStruct((NW, Nt), jnp.float32),
        mesh=_MESH,
        scratch_types=[
            pltpu.VMEM((Nt,), jnp.float32),
            pltpu.VMEM((EPW,), jnp.int32),
            pltpu.SemaphoreType.DMA,
        ],
        compiler_params=_SC_PARAMS,
    )(_hist_body)
    return k(dst2)


def _spmm_body(cbn, u_hbm, src_hbm, dst_hbm, out_hbm, sidx, didx, rows,
               acc, sem0, sem1):
    c = lax.axis_index("c")
    s = lax.axis_index("s")
    npass = cbn // NC
    zeros16 = jnp.zeros((L,), jnp.float32)
    sems = (sem0, sem1)

    def zero_rows_body(i, _):
        for j in range(128 // L):
            rows[0, i, pl.ds(j * L, L)] = zeros16
        return 0

    def gather(chunk, buf):
        pltpu.async_copy(u_hbm.at[sidx.at[chunk]], rows.at[buf], sems[buf])

    def gather_wait(chunk, buf):
        pltpu.make_async_copy(u_hbm.at[sidx.at[chunk]], rows.at[buf],
                              sems[buf]).wait()

    def scatter(chunk, buf):
        pltpu.sync_copy(rows.at[buf], acc.at[didx.at[chunk]], add=True)

    def stage(q, cb):
        pltpu.sync_copy(src_hbm.at[cb * NS * 2 + s * 2 + q], sidx)
        pltpu.sync_copy(dst_hbm.at[s * 2 + q], didx)

    def pipeline(be, bo):
        def edge_body(i2, _):
            a = 2 * i2
            gather(a + 1, bo)
            gather_wait(a, be)
            scatter(a, be)

            @pl.when(a + 2 < NCH // 2)
            def _():
                gather(a + 2, be)
            gather_wait(a + 1, bo)
            scatter(a + 1, bo)
            return 0
        lax.fori_loop(0, NCH // 4, edge_body, 0)

    def drain(cb):
        r0 = s * ROWS_PT
        pltpu.sync_copy(acc.at[pl.ds(r0, ROWS_PT)],
                        out_hbm.at[pl.ds(cb * Nt + r0, ROWS_PT)])

    for p in range(npass):
        cb = c * npass + p
        stage(0, cb)
        gather(0, 1)
        lax.fori_loop(0, CHUNK, zero_rows_body, 0)
        if p > 0:
            drain(cb - 1)
        for z in range(ROWS_PT // CHUNK):
            pltpu.sync_copy(rows.at[0],
                            acc.at[pl.ds(s * ROWS_PT + z * CHUNK, CHUNK)])
        plsc.subcore_barrier()
        pipeline(1, 0)
        stage(1, cb)
        gather(0, 0)
        pipeline(0, 1)
        plsc.subcore_barrier()

    drain(c * npass + npass - 1)


def _sc_spmm(u2d, src3, dst3, cbn):
    k = functools.partial(
        pl.kernel,
        out_type=jax.ShapeDtypeStruct((cbn * Nt, 128), jnp.float32),
        mesh=_MESH,
        scratch_types=[
            pltpu.VMEM((NCH // 2, CHUNK), jnp.int32),
            pltpu.VMEM((NCH // 2, CHUNK), jnp.int32),
            pltpu.VMEM((2, CHUNK, 128), jnp.float32),
            pltpu.VMEM_SHARED((Nt, 128), jnp.float32),
            pltpu.SemaphoreType.DMA,
            pltpu.SemaphoreType.DMA,
        ],
        compiler_params=_SC_PARAMS,
    )(functools.partial(_spmm_body, cbn))
    return k(u2d, src3, dst3)


def _prep_kernel(hist_ref, x_ref, u_ref, dinv_ref):
    deg = jnp.sum(hist_ref[...], axis=0) + 1.0
    rb = pl.program_id(0)
    row = rb * BN + lax.broadcasted_iota(jnp.int32, (BN, 1), 0)
    dinv = jnp.where(row < N, lax.rsqrt(deg.reshape(BN, 1)), 0.0)
    dinv_ref[...] = dinv
    for cb in range(IN_CH // 128):
        u_ref[cb] = x_ref[:, cb * 128:(cb + 1) * 128] * dinv


def _tc_prep(hist, xp):
    grid = (Nt // BN,)
    return pl.pallas_call(
        _prep_kernel,
        grid=grid,
        in_specs=[
            pl.BlockSpec((NW, BN), lambda r: (0, r)),
            pl.BlockSpec((BN, IN_CH), lambda r: (r, 0)),
        ],
        out_specs=[
            pl.BlockSpec((IN_CH // 128, BN, 128), lambda r: (0, r, 0)),
            pl.BlockSpec((BN, 1), lambda r: (r, 0)),
        ],
        out_shape=[
            jax.ShapeDtypeStruct((IN_CH // 128, Nt, 128), jnp.float32),
            jax.ShapeDtypeStruct((Nt, 1), jnp.float32),
        ],
    )(hist, xp)


def _stage_kernel(kbi, kbo, residual, g_ref, u_ref, w_ref, b_ref, dinv_ref,
                  uo_ref):
    acc = jnp.zeros((BN, kbo * 128), jnp.float32)
    for kb in range(kbi):
        t = (g_ref[kb] + u_ref[kb]).astype(jnp.bfloat16)
        w = w_ref[kb * 128:(kb + 1) * 128, :].astype(jnp.bfloat16)
        acc = acc + jnp.dot(t, w, preferred_element_type=jnp.float32)
    dinv = dinv_ref[...]
    z = jnp.maximum(acc * dinv + b_ref[...], 0.0)
    for ob in range(kbo):
        un = z[:, ob * 128:(ob + 1) * 128] * dinv
        if residual:
            un = un + u_ref[ob]
        uo_ref[ob] = un


def _tc_stage(g3, u3, W, b, dinv, kbi, kbo):
    residual = kbi == kbo
    grid = (Nt // BN,)
    blk3i = pl.BlockSpec((kbi, BN, 128), lambda r: (0, r, 0))
    blk3o = pl.BlockSpec((kbo, BN, 128), lambda r: (0, r, 0))
    in_specs = [
        blk3i, blk3i,
        pl.BlockSpec((kbi * 128, kbo * 128), lambda r: (0, 0)),
        pl.BlockSpec((1, kbo * 128), lambda r: (0, 0)),
        pl.BlockSpec((BN, 1), lambda r: (r, 0)),
    ]
    return pl.pallas_call(
        functools.partial(_stage_kernel, kbi, kbo, residual),
        grid=grid,
        in_specs=in_specs,
        out_specs=blk3o,
        out_shape=jax.ShapeDtypeStruct((kbo, Nt, 128), jnp.float32),
    )(g3, u3, W, b.reshape(1, -1), dinv)


def _final_kernel(g_ref, u_ref, wmu_ref, bmu_ref, wls_ref, bls_ref, dinv_ref,
                  mu_ref, ls_ref):
    kbi = H1 // 128
    accmu = jnp.zeros((BN, H2), jnp.float32)
    accls = jnp.zeros((BN, H2), jnp.float32)
    for kb in range(kbi):
        t = (g_ref[kb] + u_ref[kb]).astype(jnp.bfloat16)
        accmu = accmu + jnp.dot(
            t, wmu_ref[kb * 128:(kb + 1) * 128, :].astype(jnp.bfloat16),
            preferred_element_type=jnp.float32)
        accls = accls + jnp.dot(
            t, wls_ref[kb * 128:(kb + 1) * 128, :].astype(jnp.bfloat16),
            preferred_element_type=jnp.float32)
    dinv = dinv_ref[...]
    mu_ref[...] = accmu * dinv + bmu_ref[...]
    ls_ref[...] = accls * dinv + bls_ref[...]


def _tc_final(g3, u3, Wmu, bmu, Wls, bls, dinv):
    kbi = H1 // 128
    grid = (Nt // BN,)
    blk3 = pl.BlockSpec((kbi, BN, 128), lambda r: (0, r, 0))
    return pl.pallas_call(
        _final_kernel,
        grid=grid,
        in_specs=[
            blk3, blk3,
            pl.BlockSpec((H1, H2), lambda r: (0, 0)),
            pl.BlockSpec((1, H2), lambda r: (0, 0)),
            pl.BlockSpec((H1, H2), lambda r: (0, 0)),
            pl.BlockSpec((1, H2), lambda r: (0, 0)),
            pl.BlockSpec((BN, 1), lambda r: (r, 0)),
        ],
        out_specs=[
            pl.BlockSpec((BN, H2), lambda r: (r, 0)),
            pl.BlockSpec((BN, H2), lambda r: (r, 0)),
        ],
        out_shape=[
            jax.ShapeDtypeStruct((Nt, H2), jnp.float32),
            jax.ShapeDtypeStruct((Nt, H2), jnp.float32),
        ],
    )(g3, u3, Wmu, bmu.reshape(1, -1), Wls, bls.reshape(1, -1), dinv)


def kernel(x, edge_index, W1, b1, Wres, bres, Wmu, bmu, Wls, bls):
    npad = EPAD - E
    pad = (N + (jnp.arange(npad, dtype=jnp.int32) % 128)).astype(jnp.int32)
    srcp = jnp.concatenate([edge_index[0], pad])
    dstp = jnp.concatenate([edge_index[1], pad])
    cbs4 = (jnp.arange(4, dtype=jnp.int32) * Nt)[:, None]
    src3_4 = (srcp[None, :] + cbs4).reshape(4 * NS * 2, NCH // 2, CHUNK)
    src3_2 = (srcp[None, :] + cbs4[:2]).reshape(2 * NS * 2, NCH // 2, CHUNK)
    dst3 = dstp.reshape(NS * 2, NCH // 2, CHUNK)
    dst2 = dstp.reshape(NW, EPW)
    xp = jnp.pad(x, ((0, Nt - N), (0, 0)))

    hist = _sc_histogram(dst2)
    u, dinv = _tc_prep(hist, xp)

    kb1 = IN_CH // 128
    kbh = H1 // 128
    g = _sc_spmm(u.reshape(kb1 * Nt, 128), src3_2, dst3, kb1).reshape(kb1, Nt, 128)
    u = _tc_stage(g, u, W1, b1, dinv, kb1, kbh)
    for _ in range(DEPTH):
        g = _sc_spmm(u.reshape(kbh * Nt, 128), src3_4, dst3, kbh).reshape(kbh, Nt, 128)
        u = _tc_stage(g, u, Wres, bres, dinv, kbh, kbh)
    g = _sc_spmm(u.reshape(kbh * Nt, 128), src3_4, dst3, kbh).reshape(kbh, Nt, 128)
    mu, ls = _tc_final(g, u, Wmu, bmu, Wls, bls, dinv)
    return mu[:N], ls[:N]
